# baseline (speedup 1.0000x reference)
import functools

import numpy as np

import jax
import jax.numpy as jnp
from jax import lax
from jax.experimental import pallas as pl
from jax.experimental.pallas import tpu as pltpu
from jax.experimental.pallas import tpu_sc as plsc

B, N, E, D, DE = 8, 4096, 65536, 128, 16
DG, DOUT, DGOUT, H, A = 64, 128, 64, 256, 2
BN, BE = B * N, B * E

NC, NS, L = 2, 16, 16
GPC = B // NC
EPS = E // NS
CE = 64
NCHUNK = EPS // CE
RPS = N // NS


def _mm_kernel(x_ref, w_ref, o_ref):
    o_ref[...] = jnp.dot(x_ref[...], w_ref[...],
                         preferred_element_type=jnp.float32)


def _mm_bias_kernel(x_ref, w_ref, b_ref, o_ref):
    o_ref[...] = jnp.dot(x_ref[...], w_ref[...],
                         preferred_element_type=jnp.float32) + b_ref[...]


def _src_offset_kernel(s_ref, o_ref):
    o_ref[...] = s_ref[...] + pl.program_id(0) * N


def _node_pool_kernel(x_ref, a_ref, wx_ref, wa_ref, b_ref, o_ref):
    i = pl.program_id(1)
    emb = jnp.dot(x_ref[0], wx_ref[...], preferred_element_type=jnp.float32)
    emb += jnp.dot(a_ref[0], wa_ref[...],
                   preferred_element_type=jnp.float32)
    emb = jnp.maximum(emb + b_ref[...], 0.0)
    s = jnp.sum(emb, axis=0)[None, None]

    @pl.when(i == 0)
    def _():
        o_ref[...] = s

    @pl.when(i != 0)
    def _():
        o_ref[...] += s


def _head_kernel(p_ref, gf_ref, wg_ref, bg_ref, w1p_ref, w1g_ref, b1_ref,
                 w2_ref, b2_ref, o_ref):
    pooled = p_ref[...] * (1.0 / N)
    eg = jnp.dot(gf_ref[...], wg_ref[...],
                 preferred_element_type=jnp.float32) + bg_ref[...]
    h = jnp.dot(pooled, w1p_ref[...], preferred_element_type=jnp.float32)
    h += jnp.dot(eg, w1g_ref[...], preferred_element_type=jnp.float32)
    h = jnp.maximum(h + b1_ref[...], 0.0)
    o_ref[...] = jnp.dot(h, w2_ref[...],
                         preferred_element_type=jnp.float32) + b2_ref[...]


def _sc_edge_body(xw_hbm, ew_hbm, src_hbm, dst_hbm, agg_hbm,
                  src_v, dst_v, xw_v0, xw_v1, ew_v0, ew_v1, out_v0, out_v1,
                  agg_sh, sg0, sg1, se0, se1, ss0, ss1):
    cid = lax.axis_index("c")
    sid = lax.axis_index("s")
    xw_bufs = (xw_v0, xw_v1)
    ew_bufs = (ew_v0, ew_v1)
    out_bufs = (out_v0, out_v1)
    gsems = (sg0, sg1)
    esems = (se0, se1)
    ssems = (ss0, ss1)

    z = jnp.zeros((L,), jnp.float32)

    for gi in range(GPC):
        g = cid * GPC + gi
        row = g * NS + sid
        pltpu.sync_copy(src_hbm.at[row], src_v)
        pltpu.sync_copy(dst_hbm.at[row], dst_v)

        def zrow(e, _):
            for d in range(DOUT // L):
                out_v0[e, pl.ds(d * L, L)] = z
            return ()

        lax.fori_loop(0, CE, zrow, (), unroll=4)
        for k in range(RPS // CE):
            pltpu.sync_copy(out_v0, agg_sh.at[pl.ds(sid * RPS + k * CE, CE)])
        plsc.subcore_barrier()

        ebase = g * E + sid * EPS
        pltpu.async_copy(xw_hbm.at[src_v.at[0]], xw_v0, sg0)
        pltpu.async_copy(ew_hbm.at[pl.ds(ebase, CE)], ew_v0, se0)

        def pair(i, _):
            for b in range(2):
                j = i * 2 + b
                nb = 1 - b

                @pl.when(j + 1 < NCHUNK)
                def _():
                    pltpu.async_copy(xw_hbm.at[src_v.at[j + 1]],
                                     xw_bufs[nb], gsems[nb])
                    pltpu.async_copy(ew_hbm.at[pl.ds(ebase + (j + 1) * CE, CE)],
                                     ew_bufs[nb], esems[nb])

                pltpu.make_async_copy(xw_hbm.at[src_v.at[j]],
                                      xw_bufs[b], gsems[b]).wait()
                pltpu.make_async_copy(ew_hbm.at[pl.ds(ebase, CE)],
                                      ew_bufs[b], esems[b]).wait()

                @pl.when(j >= 2)
                def _():
                    pltpu.make_async_copy(out_bufs[b],
                                          agg_sh.at[dst_v.at[j]],
                                          ssems[b]).wait()

                def edge(e, _):
                    for d in range(DOUT // L):
                        a = xw_bufs[b][e, pl.ds(d * L, L)]
                        c = ew_bufs[b][e, pl.ds(d * L, L)]
                        out_bufs[b][e, pl.ds(d * L, L)] = (
                            jnp.maximum(a + c, 0.0))
                    return ()

                lax.fori_loop(0, CE, edge, (), unroll=4)
                pltpu.async_copy(out_bufs[b], agg_sh.at[dst_v.at[j]],
                                 ssems[b], add=True)
            return ()

        lax.fori_loop(0, NCHUNK // 2, pair, ())
        pltpu.make_async_copy(out_v0, agg_sh.at[dst_v.at[0]], ss0).wait()
        pltpu.make_async_copy(out_v1, agg_sh.at[dst_v.at[1]], ss1).wait()
        plsc.subcore_barrier()
        pltpu.sync_copy(agg_sh.at[pl.ds(sid * RPS, RPS)],
                        agg_hbm.at[pl.ds(g * N + sid * RPS, RPS)])
        plsc.subcore_barrier()


def _sc_edge_aggregate(xw, ew, src_g, dst_l):
    mesh = plsc.VectorSubcoreMesh(core_axis_name="c", subcore_axis_name="s")
    return pl.kernel(
        _sc_edge_body,
        out_type=jax.ShapeDtypeStruct((BN, DOUT), jnp.float32),
        mesh=mesh,
        scratch_types=[
            pltpu.VMEM((NCHUNK, CE), jnp.int32),
            pltpu.VMEM((NCHUNK, CE), jnp.int32),
            pltpu.VMEM((CE, DOUT), jnp.float32),
            pltpu.VMEM((CE, DOUT), jnp.float32),
            pltpu.VMEM((CE, DOUT), jnp.float32),
            pltpu.VMEM((CE, DOUT), jnp.float32),
            pltpu.VMEM((CE, DOUT), jnp.float32),
            pltpu.VMEM((CE, DOUT), jnp.float32),
            pltpu.VMEM_SHARED((N, DOUT), jnp.float32),
            pltpu.SemaphoreType.DMA,
            pltpu.SemaphoreType.DMA,
            pltpu.SemaphoreType.DMA,
            pltpu.SemaphoreType.DMA,
            pltpu.SemaphoreType.DMA,
            pltpu.SemaphoreType.DMA,
        ],
    )(xw, ew, src_g, dst_l)


def kernel(node_features, edge_features, graph_features, edges_src,
           edges_dst, W_msg, b_msg, W_node, b_node, W_g, b_g, W1, b1,
           W2, b2):
    x = node_features.reshape(BN, D)
    e = edge_features.reshape(BE, DE)

    src_g = pl.pallas_call(
        _src_offset_kernel,
        grid=(B,),
        in_specs=[pl.BlockSpec((E,), lambda i: (i,))],
        out_specs=pl.BlockSpec((E,), lambda i: (i,)),
        out_shape=jax.ShapeDtypeStruct((BE,), jnp.int32),
    )(edges_src.reshape(BE)).reshape(B * NS, NCHUNK, CE)
    dst_l = edges_dst.reshape(B * NS, NCHUNK, CE)

    Wx, We = W_msg[:D], W_msg[D:]
    Wnx, Wna = W_node[:D], W_node[D:]

    BLK = 2048
    xw = pl.pallas_call(
        _mm_kernel,
        grid=(BN // BLK,),
        in_specs=[pl.BlockSpec((BLK, D), lambda i: (i, 0)),
                  pl.BlockSpec((D, DOUT), lambda i: (0, 0))],
        out_specs=pl.BlockSpec((BLK, DOUT), lambda i: (i, 0)),
        out_shape=jax.ShapeDtypeStruct((BN, DOUT), jnp.float32),
    )(x, Wx)

    BLK2 = 4096
    ew = pl.pallas_call(
        _mm_bias_kernel,
        grid=(BE // BLK2,),
        in_specs=[pl.BlockSpec((BLK2, DE), lambda i: (i, 0)),
                  pl.BlockSpec((DE, DOUT), lambda i: (0, 0)),
                  pl.BlockSpec((1, DOUT), lambda i: (0, 0))],
        out_specs=pl.BlockSpec((BLK2, DOUT), lambda i: (i, 0)),
        out_shape=jax.ShapeDtypeStruct((BE, DOUT), jnp.float32),
    )(e, We, b_msg.reshape(1, DOUT))

    agg = _sc_edge_aggregate(xw, ew, src_g, dst_l)

    BLK3 = 1024
    x3 = x.reshape(B, N, D)
    a3 = agg.reshape(B, N, DOUT)
    pooled = pl.pallas_call(
        _node_pool_kernel,
        grid=(B, N // BLK3),
        in_specs=[pl.BlockSpec((1, BLK3, D), lambda b, i: (b, i, 0)),
                  pl.BlockSpec((1, BLK3, DOUT), lambda b, i: (b, i, 0)),
                  pl.BlockSpec((D, DOUT), lambda b, i: (0, 0)),
                  pl.BlockSpec((DOUT, DOUT), lambda b, i: (0, 0)),
                  pl.BlockSpec((1, DOUT), lambda b, i: (0, 0))],
        out_specs=pl.BlockSpec((1, 1, DOUT), lambda b, i: (b, 0, 0)),
        out_shape=jax.ShapeDtypeStruct((B, 1, DOUT), jnp.float32),
    )(x3, a3, Wnx, Wna, b_node.reshape(1, DOUT))
    pooled = pooled.reshape(B, DOUT)

    logits = pl.pallas_call(
        _head_kernel,
        in_specs=[pl.BlockSpec((B, DOUT), lambda: (0, 0)),
                  pl.BlockSpec((B, DG), lambda: (0, 0)),
                  pl.BlockSpec((DG, DGOUT), lambda: (0, 0)),
                  pl.BlockSpec((1, DGOUT), lambda: (0, 0)),
                  pl.BlockSpec((DOUT, H), lambda: (0, 0)),
                  pl.BlockSpec((DGOUT, H), lambda: (0, 0)),
                  pl.BlockSpec((1, H), lambda: (0, 0)),
                  pl.BlockSpec((H, A), lambda: (0, 0)),
                  pl.BlockSpec((1, A), lambda: (0, 0))],
        out_specs=pl.BlockSpec((B, A), lambda: (0, 0)),
        out_shape=jax.ShapeDtypeStruct((B, A), jnp.float32),
    )(pooled, graph_features, W_g, b_g.reshape(1, DGOUT),
      W1[:DOUT], W1[DOUT:], b1.reshape(1, H), W2, b2.reshape(1, A))

    return logits


# speedup vs baseline: 1.6248x; 1.6248x over previous
import functools

import numpy as np

import jax
import jax.numpy as jnp
from jax import lax
from jax.experimental import pallas as pl
from jax.experimental.pallas import tpu as pltpu
from jax.experimental.pallas import tpu_sc as plsc

B, N, E, D, DE = 8, 4096, 65536, 128, 16
DG, DOUT, DGOUT, H, A = 64, 128, 64, 256, 2
BN, BE = B * N, B * E

NC, NS, L = 2, 16, 16
GPC = B // NC
EPS = E // NS
CE = 128
NBUF = 4
NCHUNK = EPS // CE
RPS = N // NS


def _mm_kernel(x_ref, w_ref, o_ref):
    o_ref[...] = jnp.dot(x_ref[...], w_ref[...],
                         preferred_element_type=jnp.float32)


def _mm_bias_kernel(x_ref, w_ref, b_ref, o_ref):
    o_ref[...] = jnp.dot(x_ref[...], w_ref[...],
                         preferred_element_type=jnp.float32) + b_ref[...]


def _src_offset_kernel(s_ref, o_ref):
    o_ref[...] = s_ref[...] + pl.program_id(0) * N


def _node_pool_kernel(x_ref, a_ref, wx_ref, wa_ref, b_ref, o_ref):
    i = pl.program_id(1)
    emb = jnp.dot(x_ref[0], wx_ref[...], preferred_element_type=jnp.float32)
    emb += jnp.dot(a_ref[0], wa_ref[...],
                   preferred_element_type=jnp.float32)
    emb = jnp.maximum(emb + b_ref[...], 0.0)
    s = jnp.sum(emb, axis=0)[None, None]

    @pl.when(i == 0)
    def _():
        o_ref[...] = s

    @pl.when(i != 0)
    def _():
        o_ref[...] += s


def _head_kernel(p_ref, gf_ref, wg_ref, bg_ref, w1p_ref, w1g_ref, b1_ref,
                 w2_ref, b2_ref, o_ref):
    pooled = p_ref[...] * (1.0 / N)
    eg = jnp.dot(gf_ref[...], wg_ref[...],
                 preferred_element_type=jnp.float32) + bg_ref[...]
    h = jnp.dot(pooled, w1p_ref[...], preferred_element_type=jnp.float32)
    h += jnp.dot(eg, w1g_ref[...], preferred_element_type=jnp.float32)
    h = jnp.maximum(h + b1_ref[...], 0.0)
    o_ref[...] = jnp.dot(h, w2_ref[...],
                         preferred_element_type=jnp.float32) + b2_ref[...]


def _sc_edge_body(xw_hbm, ew_hbm, src_hbm, dst_hbm, agg_hbm,
                  src_v, dst_v, m_v0, m_v1, m_v2, m_v3,
                  agg_sh, sl0, sl1, sl2, sl3, sg0, sg1, sg2, sg3,
                  ss0, ss1, ss2, ss3):
    cid = lax.axis_index("c")
    sid = lax.axis_index("s")
    bufs = (m_v0, m_v1, m_v2, m_v3)
    lsems = (sl0, sl1, sl2, sl3)
    gsems = (sg0, sg1, sg2, sg3)
    ssems = (ss0, ss1, ss2, ss3)

    z = jnp.zeros((L,), jnp.float32)

    for gi in range(GPC):
        g = cid * GPC + gi
        row = g * NS + sid
        pltpu.sync_copy(src_hbm.at[row], src_v)
        pltpu.sync_copy(dst_hbm.at[row], dst_v)

        def zrow(e, _):
            for d in range(DOUT // L):
                m_v0[e, pl.ds(d * L, L)] = z
            return ()

        lax.fori_loop(0, CE, zrow, (), unroll=4)
        for k in range(RPS // CE):
            pltpu.sync_copy(m_v0, agg_sh.at[pl.ds(sid * RPS + k * CE, CE)])
        plsc.subcore_barrier()

        ebase = g * E + sid * EPS
        pltpu.sync_copy(ew_hbm.at[pl.ds(ebase, CE)], m_v0)
        pltpu.async_copy(xw_hbm.at[src_v.at[0]], m_v0, sg0, add=True)
        pltpu.async_copy(ew_hbm.at[pl.ds(ebase + CE, CE)], m_v1, sl1)

        def ring(i, _):
            for b in range(NBUF):
                j = i * NBUF + b
                pltpu.make_async_copy(xw_hbm.at[src_v.at[j]],
                                      bufs[b], gsems[b]).wait()

                def edge(e, _):
                    for d in range(DOUT // L):
                        bufs[b][e, pl.ds(d * L, L)] = jnp.maximum(
                            bufs[b][e, pl.ds(d * L, L)], 0.0)
                    return ()

                lax.fori_loop(0, CE, edge, (), unroll=4)
                pltpu.async_copy(bufs[b], agg_sh.at[dst_v.at[j]],
                                 ssems[b], add=True)

                b2 = (b + 2) % NBUF

                @pl.when(j + 2 < NCHUNK)
                def _():
                    @pl.when(j >= 2)
                    def _():
                        pltpu.make_async_copy(
                            bufs[b2], agg_sh.at[dst_v.at[j]],
                            ssems[b2]).wait()
                    pltpu.async_copy(
                        ew_hbm.at[pl.ds(ebase + (j + 2) * CE, CE)],
                        bufs[b2], lsems[b2])

                b1 = (b + 1) % NBUF

                @pl.when(j + 1 < NCHUNK)
                def _():
                    pltpu.make_async_copy(ew_hbm.at[pl.ds(ebase, CE)],
                                          bufs[b1], lsems[b1]).wait()
                    pltpu.async_copy(xw_hbm.at[src_v.at[j + 1]],
                                     bufs[b1], gsems[b1], add=True)
            return ()

        lax.fori_loop(0, NCHUNK // NBUF, ring, ())
        for b in range(NBUF):
            pltpu.make_async_copy(bufs[b], agg_sh.at[dst_v.at[0]],
                                  ssems[b]).wait()
        plsc.subcore_barrier()
        pltpu.sync_copy(agg_sh.at[pl.ds(sid * RPS, RPS)],
                        agg_hbm.at[pl.ds(g * N + sid * RPS, RPS)])
        plsc.subcore_barrier()


def _sc_edge_aggregate(xw, ew, src_g, dst_l):
    mesh = plsc.VectorSubcoreMesh(core_axis_name="c", subcore_axis_name="s")
    return pl.kernel(
        _sc_edge_body,
        out_type=jax.ShapeDtypeStruct((BN, DOUT), jnp.float32),
        mesh=mesh,
        scratch_types=(
            [pltpu.VMEM((NCHUNK, CE), jnp.int32),
             pltpu.VMEM((NCHUNK, CE), jnp.int32)]
            + [pltpu.VMEM((CE, DOUT), jnp.float32)
               for _ in range(4)]
            + [pltpu.VMEM_SHARED((N, DOUT), jnp.float32)]
            + [pltpu.SemaphoreType.DMA] * 12
        ),
    )(xw, ew, src_g, dst_l)


def kernel(node_features, edge_features, graph_features, edges_src,
           edges_dst, W_msg, b_msg, W_node, b_node, W_g, b_g, W1, b1,
           W2, b2):
    x = node_features.reshape(BN, D)
    e = edge_features.reshape(BE, DE)

    offsets = (jnp.arange(B, dtype=edges_src.dtype) * N)[:, None]
    src_g = (edges_src + offsets).reshape(B * NS, NCHUNK, CE)
    dst_l = edges_dst.reshape(B * NS, NCHUNK, CE)

    Wx, We = W_msg[:D], W_msg[D:]
    Wnx, Wna = W_node[:D], W_node[D:]

    BLK = 2048
    xw = pl.pallas_call(
        _mm_kernel,
        grid=(BN // BLK,),
        in_specs=[pl.BlockSpec((BLK, D), lambda i: (i, 0)),
                  pl.BlockSpec((D, DOUT), lambda i: (0, 0))],
        out_specs=pl.BlockSpec((BLK, DOUT), lambda i: (i, 0)),
        out_shape=jax.ShapeDtypeStruct((BN, DOUT), jnp.float32),
    )(x, Wx)

    BLK2 = 4096
    ew = pl.pallas_call(
        _mm_bias_kernel,
        grid=(BE // BLK2,),
        in_specs=[pl.BlockSpec((BLK2, DE), lambda i: (i, 0)),
                  pl.BlockSpec((DE, DOUT), lambda i: (0, 0)),
                  pl.BlockSpec((1, DOUT), lambda i: (0, 0))],
        out_specs=pl.BlockSpec((BLK2, DOUT), lambda i: (i, 0)),
        out_shape=jax.ShapeDtypeStruct((BE, DOUT), jnp.float32),
    )(e, We, b_msg.reshape(1, DOUT))

    agg = _sc_edge_aggregate(xw, ew, src_g, dst_l)

    BLK3 = 1024
    x3 = x.reshape(B, N, D)
    a3 = agg.reshape(B, N, DOUT)
    pooled = pl.pallas_call(
        _node_pool_kernel,
        grid=(B, N // BLK3),
        in_specs=[pl.BlockSpec((1, BLK3, D), lambda b, i: (b, i, 0)),
                  pl.BlockSpec((1, BLK3, DOUT), lambda b, i: (b, i, 0)),
                  pl.BlockSpec((D, DOUT), lambda b, i: (0, 0)),
                  pl.BlockSpec((DOUT, DOUT), lambda b, i: (0, 0)),
                  pl.BlockSpec((1, DOUT), lambda b, i: (0, 0))],
        out_specs=pl.BlockSpec((1, 1, DOUT), lambda b, i: (b, 0, 0)),
        out_shape=jax.ShapeDtypeStruct((B, 1, DOUT), jnp.float32),
    )(x3, a3, Wnx, Wna, b_node.reshape(1, DOUT))
    pooled = pooled.reshape(B, DOUT)

    logits = pl.pallas_call(
        _head_kernel,
        in_specs=[pl.BlockSpec((B, DOUT), lambda: (0, 0)),
                  pl.BlockSpec((B, DG), lambda: (0, 0)),
                  pl.BlockSpec((DG, DGOUT), lambda: (0, 0)),
                  pl.BlockSpec((1, DGOUT), lambda: (0, 0)),
                  pl.BlockSpec((DOUT, H), lambda: (0, 0)),
                  pl.BlockSpec((DGOUT, H), lambda: (0, 0)),
                  pl.BlockSpec((1, H), lambda: (0, 0)),
                  pl.BlockSpec((H, A), lambda: (0, 0)),
                  pl.BlockSpec((1, A), lambda: (0, 0))],
        out_specs=pl.BlockSpec((B, A), lambda: (0, 0)),
        out_shape=jax.ShapeDtypeStruct((B, A), jnp.float32),
    )(pooled, graph_features, W_g, b_g.reshape(1, DGOUT),
      W1[:DOUT], W1[DOUT:], b1.reshape(1, H), W2, b2.reshape(1, A))

    return logits


# speedup vs baseline: 1.6267x; 1.0012x over previous
import functools

import numpy as np

import jax
import jax.numpy as jnp
from jax import lax
from jax.experimental import pallas as pl
from jax.experimental.pallas import tpu as pltpu
from jax.experimental.pallas import tpu_sc as plsc

B, N, E, D, DE = 8, 4096, 65536, 128, 16
DG, DOUT, DGOUT, H, A = 64, 128, 64, 256, 2
BN, BE = B * N, B * E

NC, NS, L = 2, 16, 16
GPC = B // NC
EPS = E // NS
CE = 128
NBUF = 4
NCHUNK = EPS // CE
RPS = N // NS


def _mm_kernel(x_ref, w_ref, o_ref):
    o_ref[...] = jnp.dot(x_ref[...], w_ref[...],
                         preferred_element_type=jnp.float32)


def _mm_bias_kernel(x_ref, w_ref, b_ref, o_ref):
    o_ref[...] = jnp.dot(x_ref[...], w_ref[...],
                         preferred_element_type=jnp.float32) + b_ref[...]


def _src_offset_kernel(s_ref, o_ref):
    o_ref[...] = s_ref[...] + pl.program_id(0) * N


def _node_pool_kernel(x_ref, a_ref, wx_ref, wa_ref, b_ref, o_ref):
    i = pl.program_id(1)
    emb = jnp.dot(x_ref[0], wx_ref[...], preferred_element_type=jnp.float32)
    emb += jnp.dot(a_ref[0], wa_ref[...],
                   preferred_element_type=jnp.float32)
    emb = jnp.maximum(emb + b_ref[...], 0.0)
    s = jnp.sum(emb, axis=0)[None, None]

    @pl.when(i == 0)
    def _():
        o_ref[...] = s

    @pl.when(i != 0)
    def _():
        o_ref[...] += s


def _head_kernel(p_ref, gf_ref, wg_ref, bg_ref, w1p_ref, w1g_ref, b1_ref,
                 w2_ref, b2_ref, o_ref):
    pooled = p_ref[...] * (1.0 / N)
    eg = jnp.dot(gf_ref[...], wg_ref[...],
                 preferred_element_type=jnp.float32) + bg_ref[...]
    h = jnp.dot(pooled, w1p_ref[...], preferred_element_type=jnp.float32)
    h += jnp.dot(eg, w1g_ref[...], preferred_element_type=jnp.float32)
    h = jnp.maximum(h + b1_ref[...], 0.0)
    o_ref[...] = jnp.dot(h, w2_ref[...],
                         preferred_element_type=jnp.float32) + b2_ref[...]


def _sc_edge_body(xw_hbm, ew_hbm, src_hbm, dst_hbm, agg_hbm,
                  src_v, dst_v, m_v0, m_v1, m_v2, m_v3,
                  agg_sh, sl0, sl1, sl2, sl3, sg0, sg1, sg2, sg3,
                  ss0, ss1, ss2, ss3):
    cid = lax.axis_index("c")
    sid = lax.axis_index("s")
    bufs = (m_v0, m_v1, m_v2, m_v3)
    lsems = (sl0, sl1, sl2, sl3)
    gsems = (sg0, sg1, sg2, sg3)
    ssems = (ss0, ss1, ss2, ss3)

    z = jnp.zeros((L,), jnp.float32)

    for gi in range(GPC):
        g = cid * GPC + gi
        row = g * NS + sid
        pltpu.sync_copy(src_hbm.at[row], src_v)
        pltpu.sync_copy(dst_hbm.at[row], dst_v)

        def zrow(e, _):
            for d in range(DOUT // L):
                m_v0[e, pl.ds(d * L, L)] = z
            return ()

        lax.fori_loop(0, CE, zrow, ())
        for k in range(RPS // CE):
            pltpu.sync_copy(m_v0, agg_sh.at[pl.ds(sid * RPS + k * CE, CE)])
        plsc.subcore_barrier()

        ebase = g * E + sid * EPS
        pltpu.sync_copy(ew_hbm.at[pl.ds(ebase, CE)], m_v0)
        pltpu.async_copy(xw_hbm.at[src_v.at[0]], m_v0, sg0, add=True)
        pltpu.async_copy(ew_hbm.at[pl.ds(ebase + CE, CE)], m_v1, sl1)

        def ring(i, _):
            for b in range(NBUF):
                j = i * NBUF + b
                pltpu.make_async_copy(xw_hbm.at[src_v.at[j]],
                                      bufs[b], gsems[b]).wait()

                def edge(e, _):
                    for d in range(DOUT // L):
                        bufs[b][e, pl.ds(d * L, L)] = jnp.maximum(
                            bufs[b][e, pl.ds(d * L, L)], 0.0)
                    return ()

                lax.fori_loop(0, CE, edge, ())
                pltpu.async_copy(bufs[b], agg_sh.at[dst_v.at[j]],
                                 ssems[b], add=True)

                b2 = (b + 2) % NBUF

                @pl.when(j + 2 < NCHUNK)
                def _():
                    @pl.when(j >= 2)
                    def _():
                        pltpu.make_async_copy(
                            bufs[b2], agg_sh.at[dst_v.at[j]],
                            ssems[b2]).wait()
                    pltpu.async_copy(
                        ew_hbm.at[pl.ds(ebase + (j + 2) * CE, CE)],
                        bufs[b2], lsems[b2])

                b1 = (b + 1) % NBUF

                @pl.when(j + 1 < NCHUNK)
                def _():
                    pltpu.make_async_copy(ew_hbm.at[pl.ds(ebase, CE)],
                                          bufs[b1], lsems[b1]).wait()
                    pltpu.async_copy(xw_hbm.at[src_v.at[j + 1]],
                                     bufs[b1], gsems[b1], add=True)
            return ()

        lax.fori_loop(0, NCHUNK // NBUF, ring, ())
        for b in range(NBUF):
            pltpu.make_async_copy(bufs[b], agg_sh.at[dst_v.at[0]],
                                  ssems[b]).wait()
        plsc.subcore_barrier()
        pltpu.sync_copy(agg_sh.at[pl.ds(sid * RPS, RPS)],
                        agg_hbm.at[pl.ds(g * N + sid * RPS, RPS)])
        plsc.subcore_barrier()


def _sc_edge_aggregate(xw, ew, src_g, dst_l):
    mesh = plsc.VectorSubcoreMesh(core_axis_name="c", subcore_axis_name="s")
    return pl.kernel(
        _sc_edge_body,
        out_type=jax.ShapeDtypeStruct((BN, DOUT), jnp.float32),
        mesh=mesh,
        scratch_types=(
            [pltpu.VMEM((NCHUNK, CE), jnp.int32),
             pltpu.VMEM((NCHUNK, CE), jnp.int32)]
            + [pltpu.VMEM((CE, DOUT), jnp.float32)
               for _ in range(4)]
            + [pltpu.VMEM_SHARED((N, DOUT), jnp.float32)]
            + [pltpu.SemaphoreType.DMA] * 12
        ),
    )(xw, ew, src_g, dst_l)


def kernel(node_features, edge_features, graph_features, edges_src,
           edges_dst, W_msg, b_msg, W_node, b_node, W_g, b_g, W1, b1,
           W2, b2):
    x = node_features.reshape(BN, D)
    e = edge_features.reshape(BE, DE)

    offsets = (jnp.arange(B, dtype=edges_src.dtype) * N)[:, None]
    src_g = (edges_src + offsets).reshape(B * NS, NCHUNK, CE)
    dst_l = edges_dst.reshape(B * NS, NCHUNK, CE)

    Wx, We = W_msg[:D], W_msg[D:]
    Wnx, Wna = W_node[:D], W_node[D:]

    BLK = 2048
    xw = pl.pallas_call(
        _mm_kernel,
        grid=(BN // BLK,),
        in_specs=[pl.BlockSpec((BLK, D), lambda i: (i, 0)),
                  pl.BlockSpec((D, DOUT), lambda i: (0, 0))],
        out_specs=pl.BlockSpec((BLK, DOUT), lambda i: (i, 0)),
        out_shape=jax.ShapeDtypeStruct((BN, DOUT), jnp.float32),
    )(x, Wx)

    BLK2 = 4096
    ew = pl.pallas_call(
        _mm_bias_kernel,
        grid=(BE // BLK2,),
        in_specs=[pl.BlockSpec((BLK2, DE), lambda i: (i, 0)),
                  pl.BlockSpec((DE, DOUT), lambda i: (0, 0)),
                  pl.BlockSpec((1, DOUT), lambda i: (0, 0))],
        out_specs=pl.BlockSpec((BLK2, DOUT), lambda i: (i, 0)),
        out_shape=jax.ShapeDtypeStruct((BE, DOUT), jnp.float32),
    )(e, We, b_msg.reshape(1, DOUT))

    agg = _sc_edge_aggregate(xw, ew, src_g, dst_l)

    BLK3 = 1024
    x3 = x.reshape(B, N, D)
    a3 = agg.reshape(B, N, DOUT)
    pooled = pl.pallas_call(
        _node_pool_kernel,
        grid=(B, N // BLK3),
        in_specs=[pl.BlockSpec((1, BLK3, D), lambda b, i: (b, i, 0)),
                  pl.BlockSpec((1, BLK3, DOUT), lambda b, i: (b, i, 0)),
                  pl.BlockSpec((D, DOUT), lambda b, i: (0, 0)),
                  pl.BlockSpec((DOUT, DOUT), lambda b, i: (0, 0)),
                  pl.BlockSpec((1, DOUT), lambda b, i: (0, 0))],
        out_specs=pl.BlockSpec((1, 1, DOUT), lambda b, i: (b, 0, 0)),
        out_shape=jax.ShapeDtypeStruct((B, 1, DOUT), jnp.float32),
    )(x3, a3, Wnx, Wna, b_node.reshape(1, DOUT))
    pooled = pooled.reshape(B, DOUT)

    logits = pl.pallas_call(
        _head_kernel,
        in_specs=[pl.BlockSpec((B, DOUT), lambda: (0, 0)),
                  pl.BlockSpec((B, DG), lambda: (0, 0)),
                  pl.BlockSpec((DG, DGOUT), lambda: (0, 0)),
                  pl.BlockSpec((1, DGOUT), lambda: (0, 0)),
                  pl.BlockSpec((DOUT, H), lambda: (0, 0)),
                  pl.BlockSpec((DGOUT, H), lambda: (0, 0)),
                  pl.BlockSpec((1, H), lambda: (0, 0)),
                  pl.BlockSpec((H, A), lambda: (0, 0)),
                  pl.BlockSpec((1, A), lambda: (0, 0))],
        out_specs=pl.BlockSpec((B, A), lambda: (0, 0)),
        out_shape=jax.ShapeDtypeStruct((B, A), jnp.float32),
    )(pooled, graph_features, W_g, b_g.reshape(1, DGOUT),
      W1[:DOUT], W1[DOUT:], b1.reshape(1, H), W2, b2.reshape(1, A))

    return logits
